# Initial kernel scaffold; baseline (speedup 1.0000x reference)
import jax
import jax.numpy as jnp
from jax.experimental import pallas as pl


def kernel(lonlat):
    raise NotImplementedError("write your pallas kernel here")


# speedup vs baseline: 4.3945x; 4.3945x over previous
import math

import jax
import jax.numpy as jnp
from jax.experimental import pallas as pl

_L = 10
_NCOLS = _L * _L
_SQRT2 = math.sqrt(2.0)


def _norm_consts():
    K = {}
    for l in range(_L):
        for m in range(l + 1):
            K[(l, m)] = math.sqrt(
                (2 * l + 1) / (4.0 * math.pi)
                * math.factorial(l - m) / math.factorial(l + m)
            )
    diag = {m: -(2.0 * m - 1.0) * K[(m, m)] / K[(m - 1, m - 1)]
            for m in range(1, _L)}
    sub = {m: (2.0 * m + 1.0) * K[(m + 1, m)] / K[(m, m)]
           for m in range(_L - 1)}
    gen = {}
    for m in range(_L):
        for l in range(m + 2, _L):
            a = (2.0 * l - 1.0) / (l - m) * K[(l, m)] / K[(l - 1, m)]
            b = -(l + m - 1.0) / (l - m) * K[(l, m)] / K[(l - 2, m)]
            gen[(l, m)] = (a, b)
    return K, diag, sub, gen


_K, _DIAG, _SUB, _GEN = _norm_consts()
_DEG2RAD = math.pi / 180.0


def _sph_body(lon_ref, lat_ref, out_ref):
    lon = lon_ref[...]
    lat = lat_ref[...]
    rows, lanes = lon.shape
    npts = rows * lanes

    phi = (lon + 180.0) * _DEG2RAD
    theta = (lat + 90.0) * _DEG2RAD
    x = jnp.cos(theta)
    s = jnp.sin(theta)
    cp = jnp.cos(phi)
    sp = jnp.sin(phi)
    two_cp = cp + cp

    C = [None] * _L
    S = [None] * _L
    C[1] = _SQRT2 * cp
    S[1] = _SQRT2 * sp
    if _L > 2:
        C[2] = two_cp * C[1] - _SQRT2
        S[2] = two_cp * S[1]
    for m in range(3, _L):
        C[m] = two_cp * C[m - 1] - C[m - 2]
        S[m] = two_cp * S[m - 1] - S[m - 2]

    Q = {}
    Q[(1, 1)] = (_DIAG[1] * _K[(0, 0)]) * s
    for m in range(2, _L):
        Q[(m, m)] = _DIAG[m] * (s * Q[(m - 1, m - 1)])
    Q[(1, 0)] = (_SUB[0] * _K[(0, 0)]) * x
    for m in range(1, _L - 1):
        Q[(m + 1, m)] = _SUB[m] * (x * Q[(m, m)])
    for m in range(_L):
        for l in range(m + 2, _L):
            a, b = _GEN[(l, m)]
            if (l - 2, m) == (0, 0):
                prev2 = b * _K[(0, 0)]
                Q[(l, m)] = a * (x * Q[(l - 1, m)]) + prev2
            else:
                Q[(l, m)] = a * (x * Q[(l - 1, m)]) + b * Q[(l - 2, m)]

    cols = []
    for l in range(_L):
        for m in range(-l, l + 1):
            am = abs(m)
            if l == 0:
                cols.append(jnp.full(lon.shape, _K[(0, 0)], jnp.float32))
            elif m == 0:
                cols.append(Q[(l, 0)])
            elif m > 0:
                cols.append(C[m] * Q[(l, m)])
            else:
                cols.append(S[am] * Q[(l, am)])

    y = jnp.stack(cols, axis=1)
    y = jnp.swapaxes(y, 1, 2)
    out_ref[...] = y.reshape(npts, _NCOLS)


def kernel(lonlat):
    n = lonlat.shape[0]
    rows_per_blk = 8
    lanes = 128
    t = rows_per_blk * lanes
    lon = lonlat[:, 0].reshape(n // lanes, lanes)
    lat = lonlat[:, 1].reshape(n // lanes, lanes)
    grid = n // t
    return pl.pallas_call(
        _sph_body,
        grid=(grid,),
        in_specs=[
            pl.BlockSpec((rows_per_blk, lanes), lambda i: (i, 0)),
            pl.BlockSpec((rows_per_blk, lanes), lambda i: (i, 0)),
        ],
        out_specs=pl.BlockSpec((t, _NCOLS), lambda i: (i, 0)),
        out_shape=jax.ShapeDtypeStruct((n, _NCOLS), jnp.float32),
    )(lon, lat)


# speedup vs baseline: 7.8881x; 1.7950x over previous
import math

import jax
import jax.numpy as jnp
from jax.experimental import pallas as pl

_L = 10
_NCOLS = _L * _L
_SQRT2 = math.sqrt(2.0)


def _norm_consts():
    K = {}
    for l in range(_L):
        for m in range(l + 1):
            K[(l, m)] = math.sqrt(
                (2 * l + 1) / (4.0 * math.pi)
                * math.factorial(l - m) / math.factorial(l + m)
            )
    diag = {m: -(2.0 * m - 1.0) * K[(m, m)] / K[(m - 1, m - 1)]
            for m in range(1, _L)}
    sub = {m: (2.0 * m + 1.0) * K[(m + 1, m)] / K[(m, m)]
           for m in range(_L - 1)}
    gen = {}
    for m in range(_L):
        for l in range(m + 2, _L):
            a = (2.0 * l - 1.0) / (l - m) * K[(l, m)] / K[(l - 1, m)]
            b = -(l + m - 1.0) / (l - m) * K[(l, m)] / K[(l - 2, m)]
            gen[(l, m)] = (a, b)
    return K, diag, sub, gen


_K, _DIAG, _SUB, _GEN = _norm_consts()
_DEG2RAD = math.pi / 180.0


def _sph_body(lon_ref, lat_ref, out_ref):
    lon = lon_ref[...]
    lat = lat_ref[...]
    rows, lanes = lon.shape
    npts = rows * lanes

    phi = (lon + 180.0) * _DEG2RAD
    theta = (lat + 90.0) * _DEG2RAD
    x = jnp.cos(theta)
    s = jnp.sin(theta)
    cp = jnp.cos(phi)
    sp = jnp.sin(phi)
    two_cp = cp + cp

    C = [None] * _L
    S = [None] * _L
    C[1] = _SQRT2 * cp
    S[1] = _SQRT2 * sp
    if _L > 2:
        C[2] = two_cp * C[1] - _SQRT2
        S[2] = two_cp * S[1]
    for m in range(3, _L):
        C[m] = two_cp * C[m - 1] - C[m - 2]
        S[m] = two_cp * S[m - 1] - S[m - 2]

    Q = {}
    Q[(1, 1)] = (_DIAG[1] * _K[(0, 0)]) * s
    for m in range(2, _L):
        Q[(m, m)] = _DIAG[m] * (s * Q[(m - 1, m - 1)])
    Q[(1, 0)] = (_SUB[0] * _K[(0, 0)]) * x
    for m in range(1, _L - 1):
        Q[(m + 1, m)] = _SUB[m] * (x * Q[(m, m)])
    for m in range(_L):
        for l in range(m + 2, _L):
            a, b = _GEN[(l, m)]
            if (l - 2, m) == (0, 0):
                prev2 = b * _K[(0, 0)]
                Q[(l, m)] = a * (x * Q[(l - 1, m)]) + prev2
            else:
                Q[(l, m)] = a * (x * Q[(l - 1, m)]) + b * Q[(l - 2, m)]

    cols = []
    for l in range(_L):
        for m in range(-l, l + 1):
            am = abs(m)
            if l == 0:
                cols.append(jnp.full(lon.shape, _K[(0, 0)], jnp.float32))
            elif m == 0:
                cols.append(Q[(l, 0)])
            elif m > 0:
                cols.append(C[m] * Q[(l, m)])
            else:
                cols.append(S[am] * Q[(l, am)])

    y = jnp.stack(cols, axis=0)
    y = y.reshape(_NCOLS, npts)
    ident = jnp.eye(_NCOLS, dtype=jnp.float32)
    out_ref[...] = jax.lax.dot_general(
        y, ident, (((0,), (0,)), ((), ())),
        preferred_element_type=jnp.float32)


def kernel(lonlat):
    n = lonlat.shape[0]
    rows_per_blk = 128
    lanes = 128
    t = rows_per_blk * lanes
    lon = lonlat[:, 0].reshape(n // lanes, lanes)
    lat = lonlat[:, 1].reshape(n // lanes, lanes)
    grid = n // t
    return pl.pallas_call(
        _sph_body,
        grid=(grid,),
        in_specs=[
            pl.BlockSpec((rows_per_blk, lanes), lambda i: (i, 0)),
            pl.BlockSpec((rows_per_blk, lanes), lambda i: (i, 0)),
        ],
        out_specs=pl.BlockSpec((t, _NCOLS), lambda i: (i, 0)),
        out_shape=jax.ShapeDtypeStruct((n, _NCOLS), jnp.float32),
    )(lon, lat)
